# baseline (speedup 1.0000x reference)
import functools

import jax
import jax.numpy as jnp
from jax import lax
from jax.experimental import pallas as pl
from jax.experimental.pallas import tpu as pltpu
from jax.experimental.pallas import tpu_sc as plsc

B, N, F = 1024, 100, 16
K = 3
H = 32
D = 64
J = 16
NW = 32
NB = B // 16
BPW = NB // NW
CN = N * 16
IN_ = K * N * 16

_DOT = functools.partial(
    jnp.dot, precision=jax.lax.Precision.DEFAULT,
    preferred_element_type=jnp.float32)


def _relu(x):
    return jnp.maximum(x, 0.0)


def _sc_knn(etas_hbm, phis_hbm, out_hbm, eta_v, phi_v, idx_v):
    wid = lax.axis_index("s") * 2 + lax.axis_index("c")
    pltpu.sync_copy(etas_hbm.at[pl.ds(wid * BPW * CN, BPW * CN)], eta_v)
    pltpu.sync_copy(phis_hbm.at[pl.ds(wid * BPW * CN, BPW * CN)], phi_v)

    for b in range(BPW):
        cb = b * CN
        ib = b * IN_

        def q_body(q, carry):
            qoff = cb + q * 16
            ve = eta_v[pl.ds(qoff, 16)]
            vp = phi_v[pl.ds(qoff, 16)]

            def cand_body(j, st):
                m1, m2, m3, i1, i2, i3 = st
                joff = cb + j * 16
                ce = eta_v[pl.ds(joff, 16)]
                cp = phi_v[pl.ds(joff, 16)]
                de = ve - ce
                dp = vp - cp
                d2 = de * de + dp * dp
                jv = jnp.full((16,), j.astype(jnp.float32))
                pen = jnp.where(q == j, jnp.float32(1e9), jnp.float32(0.0))
                d2 = d2 + jnp.full((16,), pen)
                c1 = d2 < m1
                c2 = d2 < m2
                c3 = d2 < m3
                m3 = jnp.where(c3, jnp.where(c2, m2, d2), m3)
                i3 = jnp.where(c3, jnp.where(c2, i2, jv), i3)
                m2 = jnp.where(c2, jnp.where(c1, m1, d2), m2)
                i2 = jnp.where(c2, jnp.where(c1, i1, jv), i2)
                m1 = jnp.where(c1, d2, m1)
                i1 = jnp.where(c1, jv, i1)
                return m1, m2, m3, i1, i2, i3

            big = jnp.full((16,), jnp.float32(jnp.inf))
            zero = jnp.zeros((16,), jnp.float32)
            _, _, _, i1, i2, i3 = lax.fori_loop(
                0, N, cand_body, (big, big, big, zero, zero, zero),
                unroll=4)
            qo = ib + q * 16
            idx_v[pl.ds(qo, 16)] = i1
            idx_v[pl.ds(qo + N * 16, 16)] = i2
            idx_v[pl.ds(qo + 2 * N * 16, 16)] = i3
            return carry

        lax.fori_loop(0, N, q_body, 0)
    pltpu.sync_copy(idx_v, out_hbm.at[pl.ds(wid * BPW * IN_, BPW * IN_)])


def _knn_indices(etas_t, phis_t):
    mesh = plsc.VectorSubcoreMesh(core_axis_name="c", subcore_axis_name="s")
    fn = functools.partial(
        pl.kernel, mesh=mesh,
        out_type=jax.ShapeDtypeStruct((NB * IN_,), jnp.float32),
        scratch_types=[
            pltpu.VMEM((BPW * CN,), jnp.float32),
            pltpu.VMEM((BPW * CN,), jnp.float32),
            pltpu.VMEM((BPW * IN_,), jnp.float32),
        ],
    )(_sc_knn)
    return fn(etas_t, phis_t)


def _tc_pre_body(x_ref, W1c_ref, W1b_ref, b1_ref, A_ref, Bv_ref):
    xf = x_ref[...].reshape(J * N, F)
    A_ref[...] = (_DOT(xf, W1c_ref[...]) + b1_ref[...]).reshape(J, N, H)
    Bv_ref[...] = _DOT(xf, W1b_ref[...]).reshape(J, N, H)


def _tc_body(idx_ref, A_ref, Bv_ref, W2_ref,
             b2_ref, W3_ref, b3_ref, Wd1_ref, bd1_ref, Wd2_ref, bd2_ref,
             out_ref):
    iota_c = lax.broadcasted_iota(jnp.int32, (J, N, N), 2)
    iota_f = iota_c.astype(jnp.float32)
    idx = idx_ref[...]

    A = A_ref[...].reshape(J * N, H)
    Bv = Bv_ref[...]
    W2 = W2_ref[...]
    b2 = b2_ref[...]
    W3 = W3_ref[...]
    b3 = b3_ref[...]
    Wd1 = Wd1_ref[...]
    bd1 = bd1_ref[...]
    Wd2 = Wd2_ref[...]
    bd2 = bd2_ref[...]

    pt_sum = jnp.zeros((J * N, H), jnp.float32)
    for k in range(K):
        oh = (iota_f == idx[:, k, :][:, :, None]).astype(jnp.float32)
        g = lax.dot_general(
            oh, Bv, (((2,), (1,)), ((0,), (0,))),
            preferred_element_type=jnp.float32)
        h = _relu(A + g.reshape(J * N, H))
        h = _relu(_DOT(h, W2) + b2)
        h = _relu(_DOT(h, W3) + b3)
        pt_sum = pt_sum + h
    pt = pt_sum.reshape(J, N, H) * jnp.float32(1.0 / K)
    pooled = jnp.sum(pt, axis=1) * jnp.float32(1.0 / N)
    o = _relu(_DOT(pooled, Wd1) + bd1)
    o = _relu(_DOT(o, Wd2) + bd2)
    out_ref[...] = o


def kernel(inputs, W1, b1, W2, b2, W3, b3, Wd1, bd1, Wd2, bd2):
    coords = inputs[:, :, 1:3]
    eta_t = coords[:, :, 0].reshape(NB, 16, N).transpose(0, 2, 1).reshape(-1)
    phi_t = coords[:, :, 1].reshape(NB, 16, N).transpose(0, 2, 1).reshape(-1)
    raw = _knn_indices(eta_t, phi_t)
    idx = raw.reshape(NB, K, N, 16).transpose(0, 3, 1, 2).reshape(B, K, N)

    W1c = W1[:F] - W1[F:]
    W1b = W1[F:]
    full = lambda shape: pl.BlockSpec(shape, lambda i: (0,) * len(shape))
    A, Bv = pl.pallas_call(
        _tc_pre_body,
        grid=(B // J,),
        in_specs=[
            pl.BlockSpec((J, N, F), lambda i: (i, 0, 0)),
            full((F, H)), full((F, H)), full((1, H)),
        ],
        out_specs=[
            pl.BlockSpec((J, N, H), lambda i: (i, 0, 0)),
            pl.BlockSpec((J, N, H), lambda i: (i, 0, 0)),
        ],
        out_shape=[
            jax.ShapeDtypeStruct((B, N, H), jnp.float32),
            jax.ShapeDtypeStruct((B, N, H), jnp.float32),
        ],
        compiler_params=pltpu.CompilerParams(
            dimension_semantics=("arbitrary",)),
    )(inputs, W1c, W1b, b1.reshape(1, H))
    out = pl.pallas_call(
        _tc_body,
        grid=(B // J,),
        in_specs=[
            pl.BlockSpec((J, K, N), lambda i: (i, 0, 0)),
            pl.BlockSpec((J, N, H), lambda i: (i, 0, 0)),
            pl.BlockSpec((J, N, H), lambda i: (i, 0, 0)),
            full((H, H)), full((1, H)),
            full((H, H)), full((1, H)),
            full((H, D)), full((1, D)),
            full((D, D)), full((1, D)),
        ],
        out_specs=pl.BlockSpec((J, D), lambda i: (i, 0)),
        out_shape=jax.ShapeDtypeStruct((B, D), jnp.float32),
        compiler_params=pltpu.CompilerParams(
            dimension_semantics=("arbitrary",)),
    )(idx, A, Bv, W2, b2.reshape(1, H),
      W3, b3.reshape(1, H), Wd1, bd1.reshape(1, D), Wd2, bd2.reshape(1, D))
    return out


# speedup vs baseline: 1.1669x; 1.1669x over previous
import functools

import jax
import jax.numpy as jnp
from jax import lax
from jax.experimental import pallas as pl
from jax.experimental.pallas import tpu as pltpu
from jax.experimental.pallas import tpu_sc as plsc

B, N, F = 1024, 100, 16
K = 3
H = 32
D = 64
J = 16
NW = 32
NB = B // 16
BPW = NB // NW
CN = N * 16
IN_ = K * N * 16

_DG = functools.partial(
    lax.dot_general, preferred_element_type=jnp.float32)


def _relu(x):
    return jnp.maximum(x, 0.0)


def _sc_knn(etas_hbm, phis_hbm, out_hbm, eta_v, phi_v, idx_v):
    wid = lax.axis_index("s") * 2 + lax.axis_index("c")
    pltpu.sync_copy(etas_hbm.at[pl.ds(wid * BPW * CN, BPW * CN)], eta_v)
    pltpu.sync_copy(phis_hbm.at[pl.ds(wid * BPW * CN, BPW * CN)], phi_v)

    for b in range(BPW):
        cb = b * CN
        ib = b * IN_

        def q_body(q, carry):
            qoff = cb + q * 16
            ve = eta_v[pl.ds(qoff, 16)]
            vp = phi_v[pl.ds(qoff, 16)]

            def cand_body(j, st):
                m1, m2, m3, i1, i2, i3 = st
                joff = cb + j * 16
                ce = eta_v[pl.ds(joff, 16)]
                cp = phi_v[pl.ds(joff, 16)]
                de = ve - ce
                dp = vp - cp
                d2 = de * de + dp * dp
                jv = jnp.full((16,), j.astype(jnp.float32))
                pen = jnp.where(q == j, jnp.float32(1e9), jnp.float32(0.0))
                d2 = d2 + jnp.full((16,), pen)
                c1 = d2 < m1
                c2 = d2 < m2
                c3 = d2 < m3
                m3 = jnp.where(c3, jnp.where(c2, m2, d2), m3)
                i3 = jnp.where(c3, jnp.where(c2, i2, jv), i3)
                m2 = jnp.where(c2, jnp.where(c1, m1, d2), m2)
                i2 = jnp.where(c2, jnp.where(c1, i1, jv), i2)
                m1 = jnp.where(c1, d2, m1)
                i1 = jnp.where(c1, jv, i1)
                return m1, m2, m3, i1, i2, i3

            big = jnp.full((16,), jnp.float32(jnp.inf))
            zero = jnp.zeros((16,), jnp.float32)
            _, _, _, i1, i2, i3 = lax.fori_loop(
                0, N, cand_body, (big, big, big, zero, zero, zero),
                unroll=4)
            qo = ib + q * 16
            idx_v[pl.ds(qo, 16)] = i1
            idx_v[pl.ds(qo + N * 16, 16)] = i2
            idx_v[pl.ds(qo + 2 * N * 16, 16)] = i3
            return carry

        lax.fori_loop(0, N, q_body, 0)
    pltpu.sync_copy(idx_v, out_hbm.at[pl.ds(wid * BPW * IN_, BPW * IN_)])


def _knn_indices(etas_t, phis_t):
    mesh = plsc.VectorSubcoreMesh(core_axis_name="c", subcore_axis_name="s")
    fn = functools.partial(
        pl.kernel, mesh=mesh,
        out_type=jax.ShapeDtypeStruct((NB * IN_,), jnp.float32),
        scratch_types=[
            pltpu.VMEM((BPW * CN,), jnp.float32),
            pltpu.VMEM((BPW * CN,), jnp.float32),
            pltpu.VMEM((BPW * IN_,), jnp.float32),
        ],
    )(_sc_knn)
    return fn(etas_t, phis_t)


def _tc_body(idx_ref, xt_ref, W1cT_ref, W1bT_ref, b1_ref, W2T_ref,
             b2_ref, W3T_ref, b3_ref, Wd1_ref, bd1_ref, Wd2_ref, bd2_ref,
             out_ref):
    iota_n = lax.broadcasted_iota(jnp.int32, (J, N, N), 1).astype(jnp.float32)
    idx = idx_ref[...]
    xt = xt_ref[...]

    bj = lambda w: jnp.broadcast_to(w[None], (J,) + w.shape)
    W1cT = bj(W1cT_ref[...])
    W1bT = bj(W1bT_ref[...])
    W2T = bj(W2T_ref[...])
    W3T = bj(W3T_ref[...])
    b1 = b1_ref[...][None, :, :]
    b2 = b2_ref[...][None, :, :]
    b3 = b3_ref[...][None, :, :]

    bdims = (((2,), (1,)), ((0,), (0,)))
    A = _DG(W1cT, xt, bdims) + b1
    Bv = _DG(W1bT, xt, bdims)

    pt_sum = jnp.zeros((J, H, N), jnp.float32)
    for k in range(K):
        ohT = (iota_n == idx[:, k, :][:, None, :]).astype(jnp.float32)
        g = _DG(Bv, ohT, bdims)
        h = _relu(A + g)
        h = _relu(_DG(W2T, h, bdims) + b2)
        h = _relu(_DG(W3T, h, bdims) + b3)
        pt_sum = pt_sum + h
    ones_n = jnp.full((J, N, 1), jnp.float32(1.0 / (K * N)))
    pooled = _DG(pt_sum, ones_n, bdims).reshape(J, H)
    o = _relu(jnp.dot(pooled, Wd1_ref[...],
                      preferred_element_type=jnp.float32) + bd1_ref[...])
    o = _relu(jnp.dot(o, Wd2_ref[...],
                      preferred_element_type=jnp.float32) + bd2_ref[...])
    out_ref[...] = o


def kernel(inputs, W1, b1, W2, b2, W3, b3, Wd1, bd1, Wd2, bd2):
    coords = inputs[:, :, 1:3]
    eta_t = coords[:, :, 0].reshape(NB, 16, N).transpose(0, 2, 1).reshape(-1)
    phi_t = coords[:, :, 1].reshape(NB, 16, N).transpose(0, 2, 1).reshape(-1)
    raw = _knn_indices(eta_t, phi_t)
    idx = raw.reshape(NB, K, N, 16).transpose(0, 3, 1, 2).reshape(B, K, N)

    xt = inputs.transpose(0, 2, 1)
    W1cT = (W1[:F] - W1[F:]).T
    W1bT = W1[F:].T
    full = lambda shape: pl.BlockSpec(shape, lambda i: (0,) * len(shape))
    out = pl.pallas_call(
        _tc_body,
        grid=(B // J,),
        in_specs=[
            pl.BlockSpec((J, K, N), lambda i: (i, 0, 0)),
            pl.BlockSpec((J, F, N), lambda i: (i, 0, 0)),
            full((H, F)), full((H, F)), full((H, 1)),
            full((H, H)), full((H, 1)),
            full((H, H)), full((H, 1)),
            full((H, D)), full((1, D)),
            full((D, D)), full((1, D)),
        ],
        out_specs=pl.BlockSpec((J, D), lambda i: (i, 0)),
        out_shape=jax.ShapeDtypeStruct((B, D), jnp.float32),
        compiler_params=pltpu.CompilerParams(
            dimension_semantics=("arbitrary",)),
    )(idx, xt, W1cT, W1bT, b1.reshape(H, 1), W2.T, b2.reshape(H, 1),
      W3.T, b3.reshape(H, 1), Wd1, bd1.reshape(1, D), Wd2, bd2.reshape(1, D))
    return out


# speedup vs baseline: 1.3798x; 1.1825x over previous
import functools

import jax
import jax.numpy as jnp
from jax import lax
from jax.experimental import pallas as pl
from jax.experimental.pallas import tpu as pltpu
from jax.experimental.pallas import tpu_sc as plsc

B, N, F = 1024, 100, 16
K = 3
H = 32
D = 64
J = 32
NW = 32
NB = B // 16
BPW = NB // NW
CN = N * 16
IN_ = K * N * 16

_DG = functools.partial(
    lax.dot_general, preferred_element_type=jnp.float32)


def _relu(x):
    return jnp.maximum(x, 0.0)


def _sc_knn(etas_hbm, phis_hbm, out_hbm, eta_v, phi_v, idx_v):
    wid = lax.axis_index("s") * 2 + lax.axis_index("c")
    pltpu.sync_copy(etas_hbm.at[pl.ds(wid * BPW * CN, BPW * CN)], eta_v)
    pltpu.sync_copy(phis_hbm.at[pl.ds(wid * BPW * CN, BPW * CN)], phi_v)

    for b in range(BPW):
        cb = b * CN
        ib = b * IN_

        def q_body(q, carry):
            qoff = cb + q * 16
            ve = eta_v[pl.ds(qoff, 16)]
            vp = phi_v[pl.ds(qoff, 16)]

            def cand_body(j, st):
                m1, m2, m3, i1, i2, i3 = st
                joff = cb + j * 16
                ce = eta_v[pl.ds(joff, 16)]
                cp = phi_v[pl.ds(joff, 16)]
                de = ve - ce
                dp = vp - cp
                d2 = de * de + dp * dp
                jv = jnp.full((16,), j.astype(jnp.float32))
                pen = jnp.where(q == j, jnp.float32(1e9), jnp.float32(0.0))
                d2 = d2 + jnp.full((16,), pen)
                c1 = d2 < m1
                c2 = d2 < m2
                c3 = d2 < m3
                m3 = jnp.where(c3, jnp.where(c2, m2, d2), m3)
                i3 = jnp.where(c3, jnp.where(c2, i2, jv), i3)
                m2 = jnp.where(c2, jnp.where(c1, m1, d2), m2)
                i2 = jnp.where(c2, jnp.where(c1, i1, jv), i2)
                m1 = jnp.where(c1, d2, m1)
                i1 = jnp.where(c1, jv, i1)
                return m1, m2, m3, i1, i2, i3

            big = jnp.full((16,), jnp.float32(jnp.inf))
            zero = jnp.zeros((16,), jnp.float32)
            _, _, _, i1, i2, i3 = lax.fori_loop(
                0, N, cand_body, (big, big, big, zero, zero, zero),
                unroll=4)
            qo = ib + q * 16
            idx_v[pl.ds(qo, 16)] = i1
            idx_v[pl.ds(qo + N * 16, 16)] = i2
            idx_v[pl.ds(qo + 2 * N * 16, 16)] = i3
            return carry

        lax.fori_loop(0, N, q_body, 0)
    pltpu.sync_copy(idx_v, out_hbm.at[pl.ds(wid * BPW * IN_, BPW * IN_)])


def _knn_indices(etas_t, phis_t):
    mesh = plsc.VectorSubcoreMesh(core_axis_name="c", subcore_axis_name="s")
    fn = functools.partial(
        pl.kernel, mesh=mesh,
        out_type=jax.ShapeDtypeStruct((NB * IN_,), jnp.float32),
        scratch_types=[
            pltpu.VMEM((BPW * CN,), jnp.float32),
            pltpu.VMEM((BPW * CN,), jnp.float32),
            pltpu.VMEM((BPW * IN_,), jnp.float32),
        ],
    )(_sc_knn)
    return fn(etas_t, phis_t)


def _tc_body(idx_ref, xt_ref, W1cT_ref, W1bT_ref, b1_ref, W2T_ref,
             b2_ref, W3T_ref, b3_ref, Wd1_ref, bd1_ref, Wd2_ref, bd2_ref,
             out_ref):
    iota_n = lax.broadcasted_iota(jnp.int32, (J, N, N), 1).astype(jnp.float32)
    idx = idx_ref[...]
    xt = xt_ref[...]

    bj = lambda w: jnp.broadcast_to(w[None], (J,) + w.shape)
    W1cT = bj(W1cT_ref[...])
    W1bT = bj(W1bT_ref[...])
    W2T = bj(W2T_ref[...])
    W3T = bj(W3T_ref[...])
    b1 = b1_ref[...][None, :, :]
    b2 = b2_ref[...][None, :, :]
    b3 = b3_ref[...][None, :, :]

    bdims = (((2,), (1,)), ((0,), (0,)))
    A = _DG(W1cT, xt, bdims) + b1
    Bv = _DG(W1bT, xt, bdims)

    pt_sum = jnp.zeros((J, H, N), jnp.float32)
    for k in range(K):
        ohT = (iota_n == idx[:, k, :][:, None, :]).astype(jnp.float32)
        g = _DG(Bv, ohT, bdims)
        h = _relu(A + g)
        h = _relu(_DG(W2T, h, bdims) + b2)
        h = _relu(_DG(W3T, h, bdims) + b3)
        pt_sum = pt_sum + h
    ones_n = jnp.full((J, N, 1), jnp.float32(1.0 / (K * N)))
    pooled = _DG(pt_sum, ones_n, bdims).reshape(J, H)
    o = _relu(jnp.dot(pooled, Wd1_ref[...],
                      preferred_element_type=jnp.float32) + bd1_ref[...])
    o = _relu(jnp.dot(o, Wd2_ref[...],
                      preferred_element_type=jnp.float32) + bd2_ref[...])
    out_ref[...] = o


def kernel(inputs, W1, b1, W2, b2, W3, b3, Wd1, bd1, Wd2, bd2):
    coords = inputs[:, :, 1:3]
    eta_t = coords[:, :, 0].reshape(NB, 16, N).transpose(0, 2, 1).reshape(-1)
    phi_t = coords[:, :, 1].reshape(NB, 16, N).transpose(0, 2, 1).reshape(-1)
    raw = _knn_indices(eta_t, phi_t)
    idx = raw.reshape(NB, K, N, 16).transpose(0, 3, 1, 2).reshape(B, K, N)

    xt = inputs.transpose(0, 2, 1)
    W1cT = (W1[:F] - W1[F:]).T
    W1bT = W1[F:].T
    full = lambda shape: pl.BlockSpec(shape, lambda i: (0,) * len(shape))
    out = pl.pallas_call(
        _tc_body,
        grid=(B // J,),
        in_specs=[
            pl.BlockSpec((J, K, N), lambda i: (i, 0, 0)),
            pl.BlockSpec((J, F, N), lambda i: (i, 0, 0)),
            full((H, F)), full((H, F)), full((H, 1)),
            full((H, H)), full((H, 1)),
            full((H, H)), full((H, 1)),
            full((H, D)), full((1, D)),
            full((D, D)), full((1, D)),
        ],
        out_specs=pl.BlockSpec((J, D), lambda i: (i, 0)),
        out_shape=jax.ShapeDtypeStruct((B, D), jnp.float32),
        compiler_params=pltpu.CompilerParams(
            dimension_semantics=("arbitrary",)),
    )(idx, xt, W1cT, W1bT, b1.reshape(H, 1), W2.T, b2.reshape(H, 1),
      W3.T, b3.reshape(H, 1), Wd1, bd1.reshape(1, D), Wd2, bd2.reshape(1, D))
    return out


# speedup vs baseline: 1.5231x; 1.1038x over previous
import functools

import jax
import jax.numpy as jnp
from jax import lax
from jax.experimental import pallas as pl
from jax.experimental.pallas import tpu as pltpu
from jax.experimental.pallas import tpu_sc as plsc

B, N, F = 1024, 100, 16
K = 3
H = 32
D = 64
J = 64
NW = 32
NB = B // 16
BPW = NB // NW
CN = N * 16
IN_ = K * N * 16

_DG = functools.partial(
    lax.dot_general, preferred_element_type=jnp.float32)


def _relu(x):
    return jnp.maximum(x, 0.0)


def _sc_knn(etas_hbm, phis_hbm, out_hbm, eta_v, phi_v, idx_v):
    wid = lax.axis_index("s") * 2 + lax.axis_index("c")
    pltpu.sync_copy(etas_hbm.at[pl.ds(wid * BPW * CN, BPW * CN)], eta_v)
    pltpu.sync_copy(phis_hbm.at[pl.ds(wid * BPW * CN, BPW * CN)], phi_v)

    for b in range(BPW):
        cb = b * CN
        ib = b * IN_

        def q_body(q, carry):
            qoff = cb + q * 16
            ve = eta_v[pl.ds(qoff, 16)]
            vp = phi_v[pl.ds(qoff, 16)]

            def cand_body(j, st):
                m1, m2, m3, i1, i2, i3 = st
                joff = cb + j * 16
                ce = eta_v[pl.ds(joff, 16)]
                cp = phi_v[pl.ds(joff, 16)]
                de = ve - ce
                dp = vp - cp
                d2 = de * de + dp * dp
                jv = jnp.full((16,), j.astype(jnp.float32))
                pen = jnp.where(q == j, jnp.float32(1e9), jnp.float32(0.0))
                d2 = d2 + jnp.full((16,), pen)
                c1 = d2 < m1
                c2 = d2 < m2
                c3 = d2 < m3
                m3 = jnp.where(c3, jnp.where(c2, m2, d2), m3)
                i3 = jnp.where(c3, jnp.where(c2, i2, jv), i3)
                m2 = jnp.where(c2, jnp.where(c1, m1, d2), m2)
                i2 = jnp.where(c2, jnp.where(c1, i1, jv), i2)
                m1 = jnp.where(c1, d2, m1)
                i1 = jnp.where(c1, jv, i1)
                return m1, m2, m3, i1, i2, i3

            big = jnp.full((16,), jnp.float32(jnp.inf))
            zero = jnp.zeros((16,), jnp.float32)
            _, _, _, i1, i2, i3 = lax.fori_loop(
                0, N, cand_body, (big, big, big, zero, zero, zero),
                unroll=4)
            qo = ib + q * 16
            idx_v[pl.ds(qo, 16)] = i1
            idx_v[pl.ds(qo + N * 16, 16)] = i2
            idx_v[pl.ds(qo + 2 * N * 16, 16)] = i3
            return carry

        lax.fori_loop(0, N, q_body, 0)
    pltpu.sync_copy(idx_v, out_hbm.at[pl.ds(wid * BPW * IN_, BPW * IN_)])


def _knn_indices(etas_t, phis_t):
    mesh = plsc.VectorSubcoreMesh(core_axis_name="c", subcore_axis_name="s")
    fn = functools.partial(
        pl.kernel, mesh=mesh,
        out_type=jax.ShapeDtypeStruct((NB * IN_,), jnp.float32),
        scratch_types=[
            pltpu.VMEM((BPW * CN,), jnp.float32),
            pltpu.VMEM((BPW * CN,), jnp.float32),
            pltpu.VMEM((BPW * IN_,), jnp.float32),
        ],
    )(_sc_knn)
    return fn(etas_t, phis_t)


def _tc_body(idx_ref, xt_ref, W1cT_ref, W1bT_ref, b1_ref, W2T_ref,
             b2_ref, W3T_ref, b3_ref, Wd1_ref, bd1_ref, Wd2_ref, bd2_ref,
             out_ref):
    iota_n = lax.broadcasted_iota(jnp.int32, (J, N, N), 1).astype(jnp.float32)
    idx = idx_ref[...]
    xt = xt_ref[...]

    bj = lambda w: jnp.broadcast_to(w[None], (J,) + w.shape)
    W1cT = bj(W1cT_ref[...])
    W1bT = bj(W1bT_ref[...])
    W2T = bj(W2T_ref[...])
    W3T = bj(W3T_ref[...])
    b1 = b1_ref[...][None, :, :]
    b2 = b2_ref[...][None, :, :]
    b3 = b3_ref[...][None, :, :]

    bdims = (((2,), (1,)), ((0,), (0,)))
    A = _DG(W1cT, xt, bdims) + b1
    Bv = _DG(W1bT, xt, bdims)

    pt_sum = jnp.zeros((J, H, N), jnp.float32)
    for k in range(K):
        ohT = (iota_n == idx[:, k, :][:, None, :]).astype(jnp.float32)
        g = _DG(Bv, ohT, bdims)
        h = _relu(A + g)
        h = _relu(_DG(W2T, h, bdims) + b2)
        h = _relu(_DG(W3T, h, bdims) + b3)
        pt_sum = pt_sum + h
    ones_n = jnp.full((J, N, 1), jnp.float32(1.0 / (K * N)))
    pooled = _DG(pt_sum, ones_n, bdims).reshape(J, H)
    o = _relu(jnp.dot(pooled, Wd1_ref[...],
                      preferred_element_type=jnp.float32) + bd1_ref[...])
    o = _relu(jnp.dot(o, Wd2_ref[...],
                      preferred_element_type=jnp.float32) + bd2_ref[...])
    out_ref[...] = o


def kernel(inputs, W1, b1, W2, b2, W3, b3, Wd1, bd1, Wd2, bd2):
    coords = inputs[:, :, 1:3]
    eta_t = coords[:, :, 0].reshape(NB, 16, N).transpose(0, 2, 1).reshape(-1)
    phi_t = coords[:, :, 1].reshape(NB, 16, N).transpose(0, 2, 1).reshape(-1)
    raw = _knn_indices(eta_t, phi_t)
    idx = raw.reshape(NB, K, N, 16).transpose(0, 3, 1, 2).reshape(B, K, N)

    xt = inputs.transpose(0, 2, 1)
    W1cT = (W1[:F] - W1[F:]).T
    W1bT = W1[F:].T
    full = lambda shape: pl.BlockSpec(shape, lambda i: (0,) * len(shape))
    out = pl.pallas_call(
        _tc_body,
        grid=(B // J,),
        in_specs=[
            pl.BlockSpec((J, K, N), lambda i: (i, 0, 0)),
            pl.BlockSpec((J, F, N), lambda i: (i, 0, 0)),
            full((H, F)), full((H, F)), full((H, 1)),
            full((H, H)), full((H, 1)),
            full((H, H)), full((H, 1)),
            full((H, D)), full((1, D)),
            full((D, D)), full((1, D)),
        ],
        out_specs=pl.BlockSpec((J, D), lambda i: (i, 0)),
        out_shape=jax.ShapeDtypeStruct((B, D), jnp.float32),
        compiler_params=pltpu.CompilerParams(
            dimension_semantics=("arbitrary",)),
    )(idx, xt, W1cT, W1bT, b1.reshape(H, 1), W2.T, b2.reshape(H, 1),
      W3.T, b3.reshape(H, 1), Wd1, bd1.reshape(1, D), Wd2, bd2.reshape(1, D))
    return out


# speedup vs baseline: 1.5603x; 1.0244x over previous
import functools

import jax
import jax.numpy as jnp
from jax import lax
from jax.experimental import pallas as pl
from jax.experimental.pallas import tpu as pltpu
from jax.experimental.pallas import tpu_sc as plsc

B, N, F = 1024, 100, 16
K = 3
H = 32
D = 64
J = 128
NW = 32
NB = B // 16
BPW = NB // NW
CN = N * 16
IN_ = K * N * 16

_DG = functools.partial(
    lax.dot_general, preferred_element_type=jnp.float32)


def _relu(x):
    return jnp.maximum(x, 0.0)


def _sc_knn(etas_hbm, phis_hbm, out_hbm, eta_v, phi_v, idx_v):
    wid = lax.axis_index("s") * 2 + lax.axis_index("c")
    pltpu.sync_copy(etas_hbm.at[pl.ds(wid * BPW * CN, BPW * CN)], eta_v)
    pltpu.sync_copy(phis_hbm.at[pl.ds(wid * BPW * CN, BPW * CN)], phi_v)

    for b in range(BPW):
        cb = b * CN
        ib = b * IN_

        def q_body(q, carry):
            qoff = cb + q * 16
            ve = eta_v[pl.ds(qoff, 16)]
            vp = phi_v[pl.ds(qoff, 16)]

            def cand_body(j, st):
                m1, m2, m3, i1, i2, i3 = st
                joff = cb + j * 16
                ce = eta_v[pl.ds(joff, 16)]
                cp = phi_v[pl.ds(joff, 16)]
                de = ve - ce
                dp = vp - cp
                d2 = de * de + dp * dp
                jv = jnp.full((16,), j.astype(jnp.float32))
                pen = jnp.where(q == j, jnp.float32(1e9), jnp.float32(0.0))
                d2 = d2 + jnp.full((16,), pen)
                c1 = d2 < m1
                c2 = d2 < m2
                c3 = d2 < m3
                m3 = jnp.where(c3, jnp.where(c2, m2, d2), m3)
                i3 = jnp.where(c3, jnp.where(c2, i2, jv), i3)
                m2 = jnp.where(c2, jnp.where(c1, m1, d2), m2)
                i2 = jnp.where(c2, jnp.where(c1, i1, jv), i2)
                m1 = jnp.where(c1, d2, m1)
                i1 = jnp.where(c1, jv, i1)
                return m1, m2, m3, i1, i2, i3

            big = jnp.full((16,), jnp.float32(jnp.inf))
            zero = jnp.zeros((16,), jnp.float32)
            _, _, _, i1, i2, i3 = lax.fori_loop(
                0, N, cand_body, (big, big, big, zero, zero, zero),
                unroll=4)
            qo = ib + q * 16
            idx_v[pl.ds(qo, 16)] = i1
            idx_v[pl.ds(qo + N * 16, 16)] = i2
            idx_v[pl.ds(qo + 2 * N * 16, 16)] = i3
            return carry

        lax.fori_loop(0, N, q_body, 0)
    pltpu.sync_copy(idx_v, out_hbm.at[pl.ds(wid * BPW * IN_, BPW * IN_)])


def _knn_indices(etas_t, phis_t):
    mesh = plsc.VectorSubcoreMesh(core_axis_name="c", subcore_axis_name="s")
    fn = functools.partial(
        pl.kernel, mesh=mesh,
        out_type=jax.ShapeDtypeStruct((NB * IN_,), jnp.float32),
        scratch_types=[
            pltpu.VMEM((BPW * CN,), jnp.float32),
            pltpu.VMEM((BPW * CN,), jnp.float32),
            pltpu.VMEM((BPW * IN_,), jnp.float32),
        ],
    )(_sc_knn)
    return fn(etas_t, phis_t)


def _tc_body(idx_ref, xt_ref, W1cT_ref, W1bT_ref, b1_ref, W2T_ref,
             b2_ref, W3T_ref, b3_ref, Wd1_ref, bd1_ref, Wd2_ref, bd2_ref,
             out_ref):
    iota_n = lax.broadcasted_iota(jnp.int32, (J, N, N), 1).astype(jnp.float32)
    idx = idx_ref[...]
    xt = xt_ref[...]

    bj = lambda w: jnp.broadcast_to(w[None], (J,) + w.shape)
    W1cT = bj(W1cT_ref[...])
    W1bT = bj(W1bT_ref[...])
    W2T = bj(W2T_ref[...])
    W3T = bj(W3T_ref[...])
    b1 = b1_ref[...][None, :, :]
    b2 = b2_ref[...][None, :, :]
    b3 = b3_ref[...][None, :, :]

    bdims = (((2,), (1,)), ((0,), (0,)))
    A = _DG(W1cT, xt, bdims) + b1
    Bv = _DG(W1bT, xt, bdims)

    pt_sum = jnp.zeros((J, H, N), jnp.float32)
    for k in range(K):
        ohT = (iota_n == idx[:, k, :][:, None, :]).astype(jnp.float32)
        g = _DG(Bv, ohT, bdims)
        h = _relu(A + g)
        h = _relu(_DG(W2T, h, bdims) + b2)
        h = _relu(_DG(W3T, h, bdims) + b3)
        pt_sum = pt_sum + h
    ones_n = jnp.full((J, N, 1), jnp.float32(1.0 / (K * N)))
    pooled = _DG(pt_sum, ones_n, bdims).reshape(J, H)
    o = _relu(jnp.dot(pooled, Wd1_ref[...],
                      preferred_element_type=jnp.float32) + bd1_ref[...])
    o = _relu(jnp.dot(o, Wd2_ref[...],
                      preferred_element_type=jnp.float32) + bd2_ref[...])
    out_ref[...] = o


def kernel(inputs, W1, b1, W2, b2, W3, b3, Wd1, bd1, Wd2, bd2):
    coords = inputs[:, :, 1:3]
    eta_t = coords[:, :, 0].reshape(NB, 16, N).transpose(0, 2, 1).reshape(-1)
    phi_t = coords[:, :, 1].reshape(NB, 16, N).transpose(0, 2, 1).reshape(-1)
    raw = _knn_indices(eta_t, phi_t)
    idx = raw.reshape(NB, K, N, 16).transpose(0, 3, 1, 2).reshape(B, K, N)

    xt = inputs.transpose(0, 2, 1)
    W1cT = (W1[:F] - W1[F:]).T
    W1bT = W1[F:].T
    full = lambda shape: pl.BlockSpec(shape, lambda i: (0,) * len(shape))
    out = pl.pallas_call(
        _tc_body,
        grid=(B // J,),
        in_specs=[
            pl.BlockSpec((J, K, N), lambda i: (i, 0, 0)),
            pl.BlockSpec((J, F, N), lambda i: (i, 0, 0)),
            full((H, F)), full((H, F)), full((H, 1)),
            full((H, H)), full((H, 1)),
            full((H, H)), full((H, 1)),
            full((H, D)), full((1, D)),
            full((D, D)), full((1, D)),
        ],
        out_specs=pl.BlockSpec((J, D), lambda i: (i, 0)),
        out_shape=jax.ShapeDtypeStruct((B, D), jnp.float32),
        compiler_params=pltpu.CompilerParams(
            dimension_semantics=("arbitrary",)),
    )(idx, xt, W1cT, W1bT, b1.reshape(H, 1), W2.T, b2.reshape(H, 1),
      W3.T, b3.reshape(H, 1), Wd1, bd1.reshape(1, D), Wd2, bd2.reshape(1, D))
    return out


# speedup vs baseline: 1.6008x; 1.0260x over previous
import functools

import jax
import jax.numpy as jnp
from jax import lax
from jax.experimental import pallas as pl
from jax.experimental.pallas import tpu as pltpu
from jax.experimental.pallas import tpu_sc as plsc

B, N, F = 1024, 100, 16
K = 3
H = 32
D = 64
J = 128
NW = 32
NB = B // 16
BPW = NB // NW
CN = N * 16
IN_ = K * N * 16

_DG = functools.partial(
    lax.dot_general, preferred_element_type=jnp.float32)


def _relu(x):
    return jnp.maximum(x, 0.0)


def _sc_knn(etas_hbm, phis_hbm, out_hbm, eta_v, phi_v, idx_v):
    wid = lax.axis_index("s") * 2 + lax.axis_index("c")
    pltpu.sync_copy(etas_hbm.at[pl.ds(wid * BPW * CN, BPW * CN)], eta_v)
    pltpu.sync_copy(phis_hbm.at[pl.ds(wid * BPW * CN, BPW * CN)], phi_v)

    for b in range(BPW):
        cb = b * CN
        ib = b * IN_

        def q_body(q, carry):
            qoff = cb + q * 16
            ve = eta_v[pl.ds(qoff, 16)]
            vp = phi_v[pl.ds(qoff, 16)]

            def cand_body(j, st):
                m1, m2, m3, i1, i2, i3 = st
                joff = cb + j * 16
                ce = eta_v[pl.ds(joff, 16)]
                cp = phi_v[pl.ds(joff, 16)]
                de = ve - ce
                dp = vp - cp
                d2 = de * de + dp * dp
                jv = jnp.full((16,), j.astype(jnp.float32))
                pen = jnp.where(q == j, jnp.float32(1e9), jnp.float32(0.0))
                d2 = d2 + jnp.full((16,), pen)
                c1 = d2 < m1
                c2 = d2 < m2
                c3 = d2 < m3
                m3 = jnp.where(c3, jnp.where(c2, m2, d2), m3)
                i3 = jnp.where(c3, jnp.where(c2, i2, jv), i3)
                m2 = jnp.where(c2, jnp.where(c1, m1, d2), m2)
                i2 = jnp.where(c2, jnp.where(c1, i1, jv), i2)
                m1 = jnp.where(c1, d2, m1)
                i1 = jnp.where(c1, jv, i1)
                return m1, m2, m3, i1, i2, i3

            big = jnp.full((16,), jnp.float32(jnp.inf))
            zero = jnp.zeros((16,), jnp.float32)
            _, _, _, i1, i2, i3 = lax.fori_loop(
                0, N, cand_body, (big, big, big, zero, zero, zero),
                unroll=4)
            qo = ib + q * 16
            idx_v[pl.ds(qo, 16)] = i1
            idx_v[pl.ds(qo + N * 16, 16)] = i2
            idx_v[pl.ds(qo + 2 * N * 16, 16)] = i3
            return carry

        lax.fori_loop(0, N, q_body, 0)
    pltpu.sync_copy(idx_v, out_hbm.at[pl.ds(wid * BPW * IN_, BPW * IN_)])


def _knn_indices(etas_t, phis_t):
    mesh = plsc.VectorSubcoreMesh(core_axis_name="c", subcore_axis_name="s")
    fn = functools.partial(
        pl.kernel, mesh=mesh,
        out_type=jax.ShapeDtypeStruct((NB * IN_,), jnp.float32),
        scratch_types=[
            pltpu.VMEM((BPW * CN,), jnp.float32),
            pltpu.VMEM((BPW * CN,), jnp.float32),
            pltpu.VMEM((BPW * IN_,), jnp.float32),
        ],
    )(_sc_knn)
    return fn(etas_t, phis_t)


def _tc_body(idx_ref, xt_ref, W1cT_ref, W1bT_ref, b1_ref, W2T_ref,
             b2_ref, W3T_ref, b3_ref, Wd1_ref, bd1_ref, Wd2_ref, bd2_ref,
             out_ref):
    iota_n = lax.broadcasted_iota(jnp.int32, (J, N, N), 1).astype(jnp.float32)
    idx = idx_ref[...].transpose(0, 3, 1, 2).reshape(J, K, N)
    xt = xt_ref[...]

    bj = lambda w: jnp.broadcast_to(w[None], (J,) + w.shape)
    W1cT = bj(W1cT_ref[...])
    W1bT = bj(W1bT_ref[...])
    W2T = bj(W2T_ref[...])
    W3T = bj(W3T_ref[...])
    b1 = b1_ref[...][None, :, :]
    b2 = b2_ref[...][None, :, :]
    b3 = b3_ref[...][None, :, :]

    bdims = (((2,), (1,)), ((0,), (0,)))
    A = _DG(W1cT, xt, bdims) + b1
    Bv = _DG(W1bT, xt, bdims)

    pt_sum = jnp.zeros((J, H, N), jnp.float32)
    for k in range(K):
        ohT = (iota_n == idx[:, k, :][:, None, :]).astype(jnp.float32)
        g = _DG(Bv, ohT, bdims)
        h = _relu(A + g)
        h = _relu(_DG(W2T, h, bdims) + b2)
        h = _relu(_DG(W3T, h, bdims) + b3)
        pt_sum = pt_sum + h
    ones_n = jnp.full((J, N, 1), jnp.float32(1.0 / (K * N)))
    pooled = _DG(pt_sum, ones_n, bdims).reshape(J, H)
    o = _relu(jnp.dot(pooled, Wd1_ref[...],
                      preferred_element_type=jnp.float32) + bd1_ref[...])
    o = _relu(jnp.dot(o, Wd2_ref[...],
                      preferred_element_type=jnp.float32) + bd2_ref[...])
    out_ref[...] = o


def kernel(inputs, W1, b1, W2, b2, W3, b3, Wd1, bd1, Wd2, bd2):
    coords = inputs[:, :, 1:3]
    eta_t = coords[:, :, 0].reshape(NB, 16, N).transpose(0, 2, 1).reshape(-1)
    phi_t = coords[:, :, 1].reshape(NB, 16, N).transpose(0, 2, 1).reshape(-1)
    raw = _knn_indices(eta_t, phi_t)
    idx = raw.reshape(NB, K, N, 16)

    xt = inputs.transpose(0, 2, 1)
    W1cT = (W1[:F] - W1[F:]).T
    W1bT = W1[F:].T
    full = lambda shape: pl.BlockSpec(shape, lambda i: (0,) * len(shape))
    out = pl.pallas_call(
        _tc_body,
        grid=(B // J,),
        in_specs=[
            pl.BlockSpec((J // 16, K, N, 16), lambda i: (i, 0, 0, 0)),
            pl.BlockSpec((J, F, N), lambda i: (i, 0, 0)),
            full((H, F)), full((H, F)), full((H, 1)),
            full((H, H)), full((H, 1)),
            full((H, H)), full((H, 1)),
            full((H, D)), full((1, D)),
            full((D, D)), full((1, D)),
        ],
        out_specs=pl.BlockSpec((J, D), lambda i: (i, 0)),
        out_shape=jax.ShapeDtypeStruct((B, D), jnp.float32),
        compiler_params=pltpu.CompilerParams(
            dimension_semantics=("arbitrary",)),
    )(idx, xt, W1cT, W1bT, b1.reshape(H, 1), W2.T, b2.reshape(H, 1),
      W3.T, b3.reshape(H, 1), Wd1, bd1.reshape(1, D), Wd2, bd2.reshape(1, D))
    return out
